# baseline (speedup 1.0000x reference)
import jax
import jax.numpy as jnp
from jax import lax
from jax.experimental import pallas as pl
from jax.experimental.pallas import tpu as pltpu
from jax.experimental.pallas import tpu_sc as plsc

_NUM_EMB = 1024
_DIM = 64
_N = 16
_PIX = 32 * 32
_ROWS = _N * _PIX
_COST = 0.25

_SC_CORES = 2
_SC_SUBCORES = 16
_SC_WORKERS = _SC_CORES * _SC_SUBCORES
_ROWS_PER_WORKER = _ROWS // _SC_WORKERS


def _vq_body(x_ref, w_ref, idx_ref, q_ref, dsum_ref):
    x = x_ref[0]
    w = w_ref[...]
    xs = jnp.sum(x * x, axis=0, keepdims=True)
    ws = jnp.sum(w * w, axis=1, keepdims=True)
    mmn = lax.dot_general(w, -2.0 * x, (((1,), (0,)), ((), ())),
                          preferred_element_type=jnp.float32)
    d = (xs + ws) + mmn
    dmin = jnp.min(d, axis=0, keepdims=True)
    cix = lax.broadcasted_iota(jnp.int32, d.shape, 0)
    idx = jnp.min(jnp.where(d == dmin, cix, _NUM_EMB), axis=0, keepdims=True)
    idx_ref[0] = idx
    enc = (cix == idx).astype(jnp.float32)
    q_ref[0] = lax.dot_general(w, enc, (((0,), (0,)), ((), ())),
                               preferred_element_type=jnp.float32)
    dsum_ref[...] = jnp.where(pl.program_id(0) == 0, 0.0, dsum_ref[...])
    dsum_ref[...] += jnp.sum(dmin).reshape(1, 1)


def _sc_gather_body(table_hbm, idx_hbm, out_hbm, idx_v, rows_v, sem):
    wid = lax.axis_index("s") * _SC_CORES + lax.axis_index("c")
    base = wid * _ROWS_PER_WORKER
    pltpu.sync_copy(idx_hbm.at[pl.ds(base, _ROWS_PER_WORKER)], idx_v)
    pltpu.async_copy(table_hbm.at[idx_v], rows_v, sem).wait()
    pltpu.sync_copy(rows_v, out_hbm.at[pl.ds(base, _ROWS_PER_WORKER)])


def kernel(inputs, W):
    xc = inputs.reshape(_N, _DIM, _PIX)
    idx3, q, dsum = pl.pallas_call(
        _vq_body,
        grid=(_N,),
        in_specs=[
            pl.BlockSpec((1, _DIM, _PIX), lambda i: (i, 0, 0)),
            pl.BlockSpec((_NUM_EMB, _DIM), lambda i: (0, 0)),
        ],
        out_specs=[
            pl.BlockSpec((1, 1, _PIX), lambda i: (i, 0, 0)),
            pl.BlockSpec((1, _DIM, _PIX), lambda i: (i, 0, 0)),
            pl.BlockSpec((1, 1), lambda i: (0, 0)),
        ],
        out_shape=[
            jax.ShapeDtypeStruct((_N, 1, _PIX), jnp.int32),
            jax.ShapeDtypeStruct((_N, _DIM, _PIX), jnp.float32),
            jax.ShapeDtypeStruct((1, 1), jnp.float32),
        ],
    )(xc, W)

    loss = (1.0 + _COST) * dsum[0, 0] / (_ROWS * _DIM)
    qst = q.reshape(_N, _DIM, 32, 32)
    return qst, loss, idx3.reshape(_ROWS, 1)


# speedup vs baseline: 1.0251x; 1.0251x over previous
import jax
import jax.numpy as jnp
from jax import lax
from jax.experimental import pallas as pl
from jax.experimental.pallas import tpu as pltpu
from jax.experimental.pallas import tpu_sc as plsc

_NUM_EMB = 1024
_DIM = 64
_N = 16
_PIX = 32 * 32
_ROWS = _N * _PIX
_COST = 0.25

_SC_CORES = 2
_SC_SUBCORES = 16
_SC_WORKERS = _SC_CORES * _SC_SUBCORES
_ROWS_PER_WORKER = _ROWS // _SC_WORKERS


_IMGS_PER_STEP = 2


def _vq_body(x_ref, w_ref, idx_ref, q_ref, dsum_ref):
    w = w_ref[...]
    ws = jnp.sum(w * w, axis=1, keepdims=True)
    dtot = jnp.zeros((1, 1), jnp.float32)
    for j in range(_IMGS_PER_STEP):
        x = x_ref[j]
        xs = jnp.sum(x * x, axis=0, keepdims=True)
        mmn = lax.dot_general(w, -2.0 * x, (((1,), (0,)), ((), ())),
                              preferred_element_type=jnp.float32)
        d = (xs + ws) + mmn
        dmin = jnp.min(d, axis=0, keepdims=True)
        cix = lax.broadcasted_iota(jnp.int32, d.shape, 0)
        idx = jnp.min(jnp.where(d == dmin, cix, _NUM_EMB), axis=0, keepdims=True)
        idx_ref[j] = idx
        enc = (cix == idx).astype(jnp.float32)
        q_ref[j] = lax.dot_general(w, enc, (((0,), (0,)), ((), ())),
                                   preferred_element_type=jnp.float32)
        dtot += jnp.sum(dmin).reshape(1, 1)
    dsum_ref[...] = jnp.where(pl.program_id(0) == 0, 0.0, dsum_ref[...]) + dtot


def _sc_gather_body(table_hbm, idx_hbm, out_hbm, idx_v, rows_v, sem):
    wid = lax.axis_index("s") * _SC_CORES + lax.axis_index("c")
    base = wid * _ROWS_PER_WORKER
    pltpu.sync_copy(idx_hbm.at[pl.ds(base, _ROWS_PER_WORKER)], idx_v)
    pltpu.async_copy(table_hbm.at[idx_v], rows_v, sem).wait()
    pltpu.sync_copy(rows_v, out_hbm.at[pl.ds(base, _ROWS_PER_WORKER)])


def kernel(inputs, W):
    xc = inputs.reshape(_N, _DIM, _PIX)
    idx3, q, dsum = pl.pallas_call(
        _vq_body,
        grid=(_N // _IMGS_PER_STEP,),
        in_specs=[
            pl.BlockSpec((_IMGS_PER_STEP, _DIM, _PIX), lambda i: (i, 0, 0)),
            pl.BlockSpec((_NUM_EMB, _DIM), lambda i: (0, 0)),
        ],
        out_specs=[
            pl.BlockSpec((_IMGS_PER_STEP, 1, _PIX), lambda i: (i, 0, 0)),
            pl.BlockSpec((_IMGS_PER_STEP, _DIM, _PIX), lambda i: (i, 0, 0)),
            pl.BlockSpec((1, 1), lambda i: (0, 0)),
        ],
        out_shape=[
            jax.ShapeDtypeStruct((_N, 1, _PIX), jnp.int32),
            jax.ShapeDtypeStruct((_N, _DIM, _PIX), jnp.float32),
            jax.ShapeDtypeStruct((1, 1), jnp.float32),
        ],
    )(xc, W)

    loss = (1.0 + _COST) * dsum[0, 0] / (_ROWS * _DIM)
    qst = q.reshape(_N, _DIM, 32, 32)
    return qst, loss, idx3.reshape(_ROWS, 1)


# speedup vs baseline: 1.1696x; 1.1409x over previous
import jax
import jax.numpy as jnp
from jax import lax
from jax.experimental import pallas as pl
from jax.experimental.pallas import tpu as pltpu
from jax.experimental.pallas import tpu_sc as plsc

_NUM_EMB = 1024
_DIM = 64
_N = 16
_PIX = 32 * 32
_ROWS = _N * _PIX
_COST = 0.25

_SC_CORES = 2
_SC_SUBCORES = 16
_SC_WORKERS = _SC_CORES * _SC_SUBCORES
_ROWS_PER_WORKER = _ROWS // _SC_WORKERS


_IMGS_PER_STEP = 2


def _vq_body(x_ref, w_ref, idx_ref, q_ref, dsum_ref):
    w = w_ref[...]
    ws = jnp.sum(w * w, axis=1, keepdims=True)
    dtot = jnp.zeros((1, 1), jnp.float32)
    for j in range(_IMGS_PER_STEP):
        x = x_ref[j]
        xs = jnp.sum(x * x, axis=0, keepdims=True)
        mmn = lax.dot_general(w, -2.0 * x, (((1,), (0,)), ((), ())),
                              preferred_element_type=jnp.float32)
        d = (xs + ws) + mmn
        dmin = jnp.min(d, axis=0, keepdims=True)
        idx = jnp.argmin(d, axis=0)[None, :]
        idx_ref[j] = idx
        cix = lax.broadcasted_iota(jnp.int32, d.shape, 0)
        enc = (cix == idx).astype(jnp.float32)
        q_ref[j] = lax.dot_general(w, enc, (((0,), (0,)), ((), ())),
                                   preferred_element_type=jnp.float32)
        dtot += jnp.sum(dmin).reshape(1, 1)
    dsum_ref[...] = jnp.where(pl.program_id(0) == 0, 0.0, dsum_ref[...]) + dtot


def _sc_gather_body(table_hbm, idx_hbm, out_hbm, idx_v, rows_v, sem):
    wid = lax.axis_index("s") * _SC_CORES + lax.axis_index("c")
    base = wid * _ROWS_PER_WORKER
    pltpu.sync_copy(idx_hbm.at[pl.ds(base, _ROWS_PER_WORKER)], idx_v)
    pltpu.async_copy(table_hbm.at[idx_v], rows_v, sem).wait()
    pltpu.sync_copy(rows_v, out_hbm.at[pl.ds(base, _ROWS_PER_WORKER)])


def kernel(inputs, W):
    xc = inputs.reshape(_N, _DIM, _PIX)
    idx3, q, dsum = pl.pallas_call(
        _vq_body,
        grid=(_N // _IMGS_PER_STEP,),
        in_specs=[
            pl.BlockSpec((_IMGS_PER_STEP, _DIM, _PIX), lambda i: (i, 0, 0)),
            pl.BlockSpec((_NUM_EMB, _DIM), lambda i: (0, 0)),
        ],
        out_specs=[
            pl.BlockSpec((_IMGS_PER_STEP, 1, _PIX), lambda i: (i, 0, 0)),
            pl.BlockSpec((_IMGS_PER_STEP, _DIM, _PIX), lambda i: (i, 0, 0)),
            pl.BlockSpec((1, 1), lambda i: (0, 0)),
        ],
        out_shape=[
            jax.ShapeDtypeStruct((_N, 1, _PIX), jnp.int32),
            jax.ShapeDtypeStruct((_N, _DIM, _PIX), jnp.float32),
            jax.ShapeDtypeStruct((1, 1), jnp.float32),
        ],
    )(xc, W)

    loss = (1.0 + _COST) * dsum[0, 0] / (_ROWS * _DIM)
    qst = q.reshape(_N, _DIM, 32, 32)
    return qst, loss, idx3.reshape(_ROWS, 1)
